# baseline (speedup 1.0000x reference)
import functools

import jax
import jax.numpy as jnp
from jax.experimental import pallas as pl
from jax.experimental.pallas import tpu as pltpu

N = 4096
H = 512
G = 4 * H
BN = 512
NB = N // BN


def _sigmoid(x):
    return 0.5 * (jnp.tanh(0.5 * x) + 1.0)


def _dot_t(a, b):
    return jax.lax.dot_general(
        a, b, dimension_numbers=(((1,), (1,)), ((), ())),
        preferred_element_type=jnp.float32)


def _hid_in_copy(hid_hbm, hin_buf, in_sems, step, slot, j):
    return pltpu.make_async_copy(
        hid_hbm.at[pl.ds(step * BN, BN), j],
        hin_buf.at[slot, j],
        in_sems.at[slot, j])


def _out_copy(out_refs, bufs, out_sems, step, slot, j):
    hout_hbm, out_hbm = out_refs
    h0n_buf, c0n_buf, hout_buf = bufs
    src = (h0n_buf.at[slot], hout_buf.at[slot, 0],
           c0n_buf.at[slot], hout_buf.at[slot, 1],
           hout_buf.at[slot, 0])[j]
    if j == 4:
        dst = out_hbm.at[pl.ds(step * BN, BN)]
    else:
        dst = hout_hbm.at[pl.ds(step * BN, BN), j]
    return pltpu.make_async_copy(src, dst, out_sems.at[slot, j])


def _lstm_kernel(x_ref, ma_ref, mb_ref, wi0_ref, wh0_ref, wi1_ref, wh1_ref,
                 bi0_ref, bh0_ref, bi1_ref, bh1_ref, hid_hbm,
                 out_hbm, hout_hbm, hin_buf, hout_buf, h0n_buf, c0n_buf,
                 wbuf, in_sems, out_sems):
    i = pl.program_id(0)
    s_i = jax.lax.rem(i, 3)
    s_prev = jax.lax.rem(i + 2, 3)
    s_next = jax.lax.rem(i + 1, 3)
    a_slot = jax.lax.rem(i, 2)
    b_slot = jax.lax.rem(i + 1, 2)
    out_refs = (hout_hbm, out_hbm)
    bufs = (h0n_buf, c0n_buf, hout_buf)

    @pl.when(i == 0)
    def _():
        for j in range(4):
            _hid_in_copy(hid_hbm, hin_buf, in_sems, 0, 0, j).start()
        wbuf[0] = wi0_ref[...].astype(jnp.bfloat16)
        wbuf[1] = wh0_ref[...].astype(jnp.bfloat16)
        wbuf[2] = wi1_ref[...].astype(jnp.bfloat16)
        wbuf[3] = wh1_ref[...].astype(jnp.bfloat16)

    @pl.when(i + 1 < NB)
    def _():
        for j in range(4):
            _hid_in_copy(hid_hbm, hin_buf, in_sems, i + 1, s_next, j).start()

    def l0_compute():
        m = ma_ref[...].astype(jnp.float32)
        h0 = hin_buf[s_i, 0] * m
        c0 = hin_buf[s_i, 2] * m
        xb = x_ref[...].astype(jnp.bfloat16)
        gates0 = (_dot_t(xb, wbuf[0])
                  + _dot_t(h0.astype(jnp.bfloat16), wbuf[1])
                  + (bi0_ref[...] + bh0_ref[...]))
        i0 = _sigmoid(gates0[:, 0 * H:1 * H])
        f0 = _sigmoid(gates0[:, 1 * H:2 * H])
        g0 = jnp.tanh(gates0[:, 2 * H:3 * H])
        o0 = _sigmoid(gates0[:, 3 * H:4 * H])
        c0n = f0 * c0 + i0 * g0
        h0n_buf[a_slot] = o0 * jnp.tanh(c0n)
        c0n_buf[a_slot] = c0n

    def l1_compute():
        m = mb_ref[...].astype(jnp.float32)
        h1 = hin_buf[s_prev, 1] * m
        c1 = hin_buf[s_prev, 3] * m
        h0n = h0n_buf[b_slot]
        gates1 = (_dot_t(h0n.astype(jnp.bfloat16), wbuf[2])
                  + _dot_t(h1.astype(jnp.bfloat16), wbuf[3])
                  + (bi1_ref[...] + bh1_ref[...]))
        i1 = _sigmoid(gates1[:, 0 * H:1 * H])
        f1 = _sigmoid(gates1[:, 1 * H:2 * H])
        g1 = jnp.tanh(gates1[:, 2 * H:3 * H])
        o1 = _sigmoid(gates1[:, 3 * H:4 * H])
        c1n = f1 * c1 + i1 * g1
        hout_buf[b_slot, 0] = o1 * jnp.tanh(c1n)
        hout_buf[b_slot, 1] = c1n
        for j in range(5):
            _out_copy(out_refs, bufs, out_sems, i - 1, b_slot, j).start()

    def wait_in():
        for j in range(4):
            _hid_in_copy(hid_hbm, hin_buf, in_sems, i, s_i, j).wait()

    def drain_a():
        @pl.when(i >= 2)
        def _():
            _out_copy(out_refs, bufs, out_sems, i - 2, a_slot, 0).wait()
            _out_copy(out_refs, bufs, out_sems, i - 2, a_slot, 2).wait()

    def drain_b():
        @pl.when(i >= 3)
        def _():
            for j in (1, 3, 4):
                _out_copy(out_refs, bufs, out_sems, i - 3, b_slot, j).wait()

    @pl.when(jnp.logical_and(i >= 1, i < NB))
    def _():
        wait_in()
        drain_a()
        drain_b()
        l0_compute()
        l1_compute()

    @pl.when(i == 0)
    def _():
        wait_in()
        l0_compute()

    @pl.when(i == NB)
    def _():
        drain_b()
        l1_compute()
        for j in range(5):
            _out_copy(out_refs, bufs, out_sems, i - 2, a_slot, j).wait()
            _out_copy(out_refs, bufs, out_sems, i - 1, b_slot, j).wait()


@functools.partial(jax.jit, static_argnames=("interpret",))
def _run(x, hs, masks, wi0, wh0, wi1, wh1, bi0, bh0, bi1, bh1,
         interpret=False):
    grid = (NB + 1,)
    cur = lambda i: (jnp.minimum(i, NB - 1), 0)
    prev = lambda i: (jnp.maximum(i - 1, 0), 0)
    rep = lambda i: (0, 0)
    out, hout = pl.pallas_call(
        _lstm_kernel,
        grid=grid,
        in_specs=[
            pl.BlockSpec((BN, H), cur),
            pl.BlockSpec((BN, 1), cur),
            pl.BlockSpec((BN, 1), prev),
            pl.BlockSpec((G, H), rep),
            pl.BlockSpec((G, H), rep),
            pl.BlockSpec((G, H), rep),
            pl.BlockSpec((G, H), rep),
            pl.BlockSpec((1, G), rep),
            pl.BlockSpec((1, G), rep),
            pl.BlockSpec((1, G), rep),
            pl.BlockSpec((1, G), rep),
            pl.BlockSpec(memory_space=pltpu.MemorySpace.HBM),
        ],
        out_specs=[
            pl.BlockSpec(memory_space=pltpu.MemorySpace.HBM),
            pl.BlockSpec(memory_space=pltpu.MemorySpace.HBM),
        ],
        out_shape=[
            jax.ShapeDtypeStruct((N, H), jnp.float32),
            jax.ShapeDtypeStruct((N, 4, H), jnp.float32),
        ],
        scratch_shapes=[
            pltpu.VMEM((3, 4, BN, H), jnp.float32),
            pltpu.VMEM((2, 2, BN, H), jnp.float32),
            pltpu.VMEM((2, BN, H), jnp.float32),
            pltpu.VMEM((2, BN, H), jnp.float32),
            pltpu.VMEM((4, G, H), jnp.bfloat16),
            pltpu.SemaphoreType.DMA((3, 4)),
            pltpu.SemaphoreType.DMA((2, 5)),
        ],
        compiler_params=pltpu.CompilerParams(
            dimension_semantics=("arbitrary",),
        ),
        interpret=interpret,
    )(x, masks, masks, wi0, wh0, wi1, wh1, bi0, bh0, bi1, bh1, hs)
    return out, hout


def kernel(x, hidden_states, masks, W_ih_0, W_hh_0, b_ih_0, b_hh_0,
           W_ih_1, W_hh_1, b_ih_1, b_hh_1, *, interpret=False):
    out, hout = _run(x, hidden_states, masks, W_ih_0, W_hh_0, W_ih_1, W_hh_1,
                     b_ih_0.reshape(1, G), b_hh_0.reshape(1, G),
                     b_ih_1.reshape(1, G), b_hh_1.reshape(1, G),
                     interpret=interpret)
    return out, hout


# speedup vs baseline: 1.1821x; 1.1821x over previous
import functools

import jax
import jax.numpy as jnp
from jax.experimental import pallas as pl
from jax.experimental.pallas import tpu as pltpu

N = 4096
H = 512
G = 4 * H
BN = 512
SPLIT = 2


def _sigmoid(x):
    return 0.5 * (jnp.tanh(0.5 * x) + 1.0)


def _dot_t(a, b):
    return jax.lax.dot_general(
        a, b, dimension_numbers=(((1,), (1,)), ((), ())),
        preferred_element_type=jnp.float32)


def _hid_in_copy(hid_hbm, hin_buf, in_sems, step, slot, j):
    return pltpu.make_async_copy(
        hid_hbm.at[pl.ds(step * BN, BN), j],
        hin_buf.at[slot, j],
        in_sems.at[slot, j])


def _hid_out_copy(hout_hbm, hout_buf, out_sems, step, slot, j):
    return pltpu.make_async_copy(
        hout_buf.at[slot, j],
        hout_hbm.at[pl.ds(step * BN, BN), j],
        out_sems.at[slot, j])


def _out_copy(out_hbm, hout_buf, out_sems, step, slot):
    return pltpu.make_async_copy(
        hout_buf.at[slot, 1],
        out_hbm.at[pl.ds(step * BN, BN)],
        out_sems.at[slot, 4])


def _cell_chain(x_ref, m_ref, hin_buf, slot, wi0, wh0, wi1, wh1, b0, b1,
                hout_buf, lo, rows):
    sub = pl.ds(lo, rows)
    m = m_ref[sub, :].astype(jnp.float32)
    h0 = hin_buf[slot, 0, sub, :] * m
    h1 = hin_buf[slot, 1, sub, :] * m
    c0 = hin_buf[slot, 2, sub, :] * m
    c1 = hin_buf[slot, 3, sub, :] * m

    xb = x_ref[sub, :].astype(jnp.bfloat16)
    gates0 = _dot_t(xb, wi0) + _dot_t(h0.astype(jnp.bfloat16), wh0) + b0
    i0 = _sigmoid(gates0[:, 0 * H:1 * H])
    f0 = _sigmoid(gates0[:, 1 * H:2 * H])
    g0 = jnp.tanh(gates0[:, 2 * H:3 * H])
    o0 = _sigmoid(gates0[:, 3 * H:4 * H])
    c0n = f0 * c0 + i0 * g0
    h0n = o0 * jnp.tanh(c0n)

    gates1 = (_dot_t(h0n.astype(jnp.bfloat16), wi1)
              + _dot_t(h1.astype(jnp.bfloat16), wh1) + b1)
    i1 = _sigmoid(gates1[:, 0 * H:1 * H])
    f1 = _sigmoid(gates1[:, 1 * H:2 * H])
    g1 = jnp.tanh(gates1[:, 2 * H:3 * H])
    o1 = _sigmoid(gates1[:, 3 * H:4 * H])
    c1n = f1 * c1 + i1 * g1
    h1n = o1 * jnp.tanh(c1n)

    hout_buf[slot, 0, sub, :] = h0n
    hout_buf[slot, 1, sub, :] = h1n
    hout_buf[slot, 2, sub, :] = c0n
    hout_buf[slot, 3, sub, :] = c1n


def _lstm_kernel(x_ref, m_ref, wi0_ref, wh0_ref, wi1_ref, wh1_ref,
                 bi0_ref, bh0_ref, bi1_ref, bh1_ref, hid_hbm,
                 out_ref, hout_hbm, hin_buf, hout_buf, wbuf, in_sems, out_sems):
    i = pl.program_id(0)
    nsteps = pl.num_programs(0)
    slot = jax.lax.rem(i, 3)

    @pl.when(i == 0)
    def _():
        for j in range(4):
            _hid_in_copy(hid_hbm, hin_buf, in_sems, 0, 0, j).start()
            _hid_in_copy(hid_hbm, hin_buf, in_sems, 1, 1, j).start()

    @pl.when(i + 2 < nsteps)
    def _():
        for j in range(4):
            _hid_in_copy(hid_hbm, hin_buf, in_sems, i + 2,
                         jax.lax.rem(i + 2, 3), j).start()

    for j in range(4):
        _hid_in_copy(hid_hbm, hin_buf, in_sems, i, slot, j).wait()

    @pl.when(i >= 3)
    def _():
        for j in range(4):
            _hid_out_copy(hout_hbm, hout_buf, out_sems, i - 3, slot, j).wait()
        _out_copy(out_ref, hout_buf, out_sems, i - 3, slot).wait()

    @pl.when(i == 0)
    def _():
        wbuf[0] = wi0_ref[...].astype(jnp.bfloat16)
        wbuf[1] = wh0_ref[...].astype(jnp.bfloat16)
        wbuf[2] = wi1_ref[...].astype(jnp.bfloat16)
        wbuf[3] = wh1_ref[...].astype(jnp.bfloat16)

    wi0 = wbuf[0]
    wh0 = wbuf[1]
    wi1 = wbuf[2]
    wh1 = wbuf[3]
    b0 = bi0_ref[...] + bh0_ref[...]
    b1 = bi1_ref[...] + bh1_ref[...]

    rows = BN // SPLIT
    for s in range(SPLIT):
        _cell_chain(x_ref, m_ref, hin_buf, slot, wi0, wh0, wi1, wh1, b0, b1,
                    hout_buf, s * rows, rows)

    for j in range(4):
        _hid_out_copy(hout_hbm, hout_buf, out_sems, i, slot, j).start()
    _out_copy(out_ref, hout_buf, out_sems, i, slot).start()

    @pl.when(i == nsteps - 1)
    def _():
        for d in range(3):
            ds_ = jax.lax.rem(i - d + 3, 3)
            for j in range(4):
                _hid_out_copy(hout_hbm, hout_buf, out_sems, i - d, ds_, j).wait()
            _out_copy(out_ref, hout_buf, out_sems, i - d, ds_).wait()


@functools.partial(jax.jit, static_argnames=("interpret",))
def _run(x, hs, mf, wi0, wh0, wi1, wh1, bi0, bh0, bi1, bh1, interpret=False):
    grid = (N // BN,)
    row = lambda i: (i, 0)
    rep = lambda i: (0, 0)
    out, hout = pl.pallas_call(
        _lstm_kernel,
        grid=grid,
        in_specs=[
            pl.BlockSpec((BN, H), row),
            pl.BlockSpec((BN, 1), row),
            pl.BlockSpec((G, H), rep),
            pl.BlockSpec((G, H), rep),
            pl.BlockSpec((G, H), rep),
            pl.BlockSpec((G, H), rep),
            pl.BlockSpec((1, G), rep),
            pl.BlockSpec((1, G), rep),
            pl.BlockSpec((1, G), rep),
            pl.BlockSpec((1, G), rep),
            pl.BlockSpec(memory_space=pltpu.MemorySpace.HBM),
        ],
        out_specs=[
            pl.BlockSpec(memory_space=pltpu.MemorySpace.HBM),
            pl.BlockSpec(memory_space=pltpu.MemorySpace.HBM),
        ],
        out_shape=[
            jax.ShapeDtypeStruct((N, H), jnp.float32),
            jax.ShapeDtypeStruct((N, 4, H), jnp.float32),
        ],
        scratch_shapes=[
            pltpu.VMEM((3, 4, BN, H), jnp.float32),
            pltpu.VMEM((3, 4, BN, H), jnp.float32),
            pltpu.VMEM((4, G, H), jnp.bfloat16),
            pltpu.SemaphoreType.DMA((3, 4)),
            pltpu.SemaphoreType.DMA((3, 5)),
        ],
        compiler_params=pltpu.CompilerParams(
            dimension_semantics=("arbitrary",),
        ),
        interpret=interpret,
    )(x, mf, wi0, wh0, wi1, wh1, bi0, bh0, bi1, bh1, hs)
    return out, hout


def kernel(x, hidden_states, masks, W_ih_0, W_hh_0, b_ih_0, b_hh_0,
           W_ih_1, W_hh_1, b_ih_1, b_hh_1, *, interpret=False):
    out, hout = _run(x, hidden_states, masks, W_ih_0, W_hh_0, W_ih_1, W_hh_1,
                     b_ih_0.reshape(1, G), b_hh_0.reshape(1, G),
                     b_ih_1.reshape(1, G), b_hh_1.reshape(1, G),
                     interpret=interpret)
    return out, hout


# speedup vs baseline: 1.2080x; 1.0219x over previous
import functools

import jax
import jax.numpy as jnp
from jax.experimental import pallas as pl
from jax.experimental.pallas import tpu as pltpu

N = 4096
H = 512
G = 4 * H
BN = 512
SPLIT = 2


def _sigmoid(x):
    return 0.5 * (jnp.tanh(0.5 * x) + 1.0)


def _dot_t(a, b):
    return jax.lax.dot_general(
        a, b, dimension_numbers=(((1,), (1,)), ((), ())),
        preferred_element_type=jnp.float32)


def _hid_in_copy(hid_hbm, hin_buf, in_sems, step, slot, j):
    return pltpu.make_async_copy(
        hid_hbm.at[pl.ds(step * BN, BN), j],
        hin_buf.at[slot, j],
        in_sems.at[slot, j])


def _hid_out_copy(hout_hbm, hout_buf, out_sems, step, slot, j):
    return pltpu.make_async_copy(
        hout_buf.at[slot, j],
        hout_hbm.at[pl.ds(step * BN, BN), j],
        out_sems.at[slot, j])


def _out_copy(out_hbm, hout_buf, out_sems, step, slot):
    return pltpu.make_async_copy(
        hout_buf.at[slot, 1],
        out_hbm.at[pl.ds(step * BN, BN)],
        out_sems.at[slot, 4])


def _cell_chain(x_ref, m_ref, hin_buf, slot, wi0, wh0, wi1, wh1, b0, b1,
                hout_buf, lo, rows):
    sub = pl.ds(lo, rows)
    m = m_ref[sub, :].astype(jnp.float32)
    h0 = hin_buf[slot, 0, sub, :] * m
    h1 = hin_buf[slot, 1, sub, :] * m
    c0 = hin_buf[slot, 2, sub, :] * m
    c1 = hin_buf[slot, 3, sub, :] * m

    xb = x_ref[sub, :].astype(jnp.bfloat16)
    gates0 = _dot_t(xb, wi0) + _dot_t(h0.astype(jnp.bfloat16), wh0) + b0
    i0 = _sigmoid(gates0[:, 0 * H:1 * H])
    f0 = _sigmoid(gates0[:, 1 * H:2 * H])
    g0 = jnp.tanh(gates0[:, 2 * H:3 * H])
    o0 = _sigmoid(gates0[:, 3 * H:4 * H])
    c0n = f0 * c0 + i0 * g0
    h0n = o0 * jnp.tanh(c0n)

    gates1 = (_dot_t(h0n.astype(jnp.bfloat16), wi1)
              + _dot_t(h1.astype(jnp.bfloat16), wh1) + b1)
    i1 = _sigmoid(gates1[:, 0 * H:1 * H])
    f1 = _sigmoid(gates1[:, 1 * H:2 * H])
    g1 = jnp.tanh(gates1[:, 2 * H:3 * H])
    o1 = _sigmoid(gates1[:, 3 * H:4 * H])
    c1n = f1 * c1 + i1 * g1
    h1n = o1 * jnp.tanh(c1n)

    hout_buf[slot, 0, sub, :] = h0n
    hout_buf[slot, 1, sub, :] = h1n
    hout_buf[slot, 2, sub, :] = c0n
    hout_buf[slot, 3, sub, :] = c1n


def _lstm_kernel(x_ref, m_ref, wi0_ref, wh0_ref, wi1_ref, wh1_ref,
                 bi0_ref, bh0_ref, bi1_ref, bh1_ref, hid_hbm,
                 out_ref, hout_hbm, hin_buf, hout_buf, wbuf, in_sems, out_sems):
    i = pl.program_id(0)
    nsteps = pl.num_programs(0)
    slot = jax.lax.rem(i, 2)
    nslot = jax.lax.rem(i + 1, 2)

    @pl.when(i == 0)
    def _():
        for j in range(4):
            _hid_in_copy(hid_hbm, hin_buf, in_sems, 0, 0, j).start()

    @pl.when(i + 1 < nsteps)
    def _():
        for j in range(4):
            _hid_in_copy(hid_hbm, hin_buf, in_sems, i + 1, nslot, j).start()

    for j in range(4):
        _hid_in_copy(hid_hbm, hin_buf, in_sems, i, slot, j).wait()

    @pl.when(i >= 2)
    def _():
        for j in range(4):
            _hid_out_copy(hout_hbm, hout_buf, out_sems, i - 2, slot, j).wait()
        _out_copy(out_ref, hout_buf, out_sems, i - 2, slot).wait()

    @pl.when(i == 0)
    def _():
        wbuf[0] = wi0_ref[...].astype(jnp.bfloat16)
        wbuf[1] = wh0_ref[...].astype(jnp.bfloat16)
        wbuf[2] = wi1_ref[...].astype(jnp.bfloat16)
        wbuf[3] = wh1_ref[...].astype(jnp.bfloat16)

    wi0 = wbuf[0]
    wh0 = wbuf[1]
    wi1 = wbuf[2]
    wh1 = wbuf[3]
    b0 = bi0_ref[...] + bh0_ref[...]
    b1 = bi1_ref[...] + bh1_ref[...]

    rows = BN // SPLIT
    for s in range(SPLIT):
        _cell_chain(x_ref, m_ref, hin_buf, slot, wi0, wh0, wi1, wh1, b0, b1,
                    hout_buf, s * rows, rows)

    for j in range(4):
        _hid_out_copy(hout_hbm, hout_buf, out_sems, i, slot, j).start()
    _out_copy(out_ref, hout_buf, out_sems, i, slot).start()

    @pl.when(i == nsteps - 1)
    def _():
        for j in range(4):
            _hid_out_copy(hout_hbm, hout_buf, out_sems, i - 1, nslot, j).wait()
            _hid_out_copy(hout_hbm, hout_buf, out_sems, i, slot, j).wait()
        _out_copy(out_ref, hout_buf, out_sems, i - 1, nslot).wait()
        _out_copy(out_ref, hout_buf, out_sems, i, slot).wait()


@functools.partial(jax.jit, static_argnames=("interpret",))
def _run(x, hs, mf, wi0, wh0, wi1, wh1, bi0, bh0, bi1, bh1, interpret=False):
    grid = (N // BN,)
    row = lambda i: (i, 0)
    rep = lambda i: (0, 0)
    out, hout = pl.pallas_call(
        _lstm_kernel,
        grid=grid,
        in_specs=[
            pl.BlockSpec((BN, H), row),
            pl.BlockSpec((BN, 1), row),
            pl.BlockSpec((G, H), rep),
            pl.BlockSpec((G, H), rep),
            pl.BlockSpec((G, H), rep),
            pl.BlockSpec((G, H), rep),
            pl.BlockSpec((1, G), rep),
            pl.BlockSpec((1, G), rep),
            pl.BlockSpec((1, G), rep),
            pl.BlockSpec((1, G), rep),
            pl.BlockSpec(memory_space=pltpu.MemorySpace.HBM),
        ],
        out_specs=[
            pl.BlockSpec(memory_space=pltpu.MemorySpace.HBM),
            pl.BlockSpec(memory_space=pltpu.MemorySpace.HBM),
        ],
        out_shape=[
            jax.ShapeDtypeStruct((N, H), jnp.float32),
            jax.ShapeDtypeStruct((N, 4, H), jnp.float32),
        ],
        scratch_shapes=[
            pltpu.VMEM((2, 4, BN, H), jnp.float32),
            pltpu.VMEM((2, 4, BN, H), jnp.float32),
            pltpu.VMEM((4, G, H), jnp.bfloat16),
            pltpu.SemaphoreType.DMA((2, 4)),
            pltpu.SemaphoreType.DMA((2, 5)),
        ],
        compiler_params=pltpu.CompilerParams(
            dimension_semantics=("arbitrary",),
        ),
        interpret=interpret,
    )(x, mf, wi0, wh0, wi1, wh1, bi0, bh0, bi1, bh1, hs)
    return out, hout


def kernel(x, hidden_states, masks, W_ih_0, W_hh_0, b_ih_0, b_hh_0,
           W_ih_1, W_hh_1, b_ih_1, b_hh_1, *, interpret=False):
    out, hout = _run(x, hidden_states, masks, W_ih_0, W_hh_0, W_ih_1, W_hh_1,
                     b_ih_0.reshape(1, G), b_hh_0.reshape(1, G),
                     b_ih_1.reshape(1, G), b_hh_1.reshape(1, G),
                     interpret=interpret)
    return out, hout
